# baseline (speedup 1.0000x reference)
import functools

import jax
import jax.numpy as jnp
from jax import lax
from jax.experimental import pallas as pl
from jax.experimental.pallas import tpu as pltpu
from jax.experimental.pallas import tpu_sc as plsc

EMB_DIM = 64
NUM_CORES = 2
NUM_SUBCORES = 16
NUM_WORKERS = NUM_CORES * NUM_SUBCORES
CHUNK = 128
NBUF = 5


@functools.lru_cache(maxsize=None)
def _build(num_idx):
    assert num_idx % (NUM_WORKERS * CHUNK) == 0
    per_worker = num_idx // NUM_WORKERS
    nchunk = per_worker // CHUNK
    assert nchunk % NBUF == 0 and nchunk >= 2 * NBUF

    mesh = plsc.VectorSubcoreMesh(core_axis_name="c", subcore_axis_name="s")

    @functools.partial(
        pl.kernel,
        out_type=jax.ShapeDtypeStruct((num_idx, EMB_DIM), jnp.float32),
        mesh=mesh,
        scratch_types=[
            pltpu.VMEM((nchunk, CHUNK), jnp.int32),
            pltpu.VMEM((NBUF, CHUNK, EMB_DIM), jnp.float32),
            pltpu.SemaphoreType.DMA((NBUF,)),
            pltpu.SemaphoreType.DMA((NBUF,)),
        ],
        compiler_params=pltpu.CompilerParams(use_tc_tiling_on_sc=False),
    )
    def emb_kernel(idx_hbm, table_hbm, out_hbm, idx_v, rows_v, gsem, ssem):
        wid = lax.axis_index("s") * NUM_CORES + lax.axis_index("c")
        row_base = wid * per_worker

        pltpu.sync_copy(idx_hbm.at[wid], idx_v)

        def start_gather(j, b):
            pltpu.async_copy(table_hbm.at[idx_v.at[j]], rows_v.at[b],
                             gsem.at[b])

        def wait_gather(b):
            pltpu.make_async_copy(table_hbm.at[idx_v.at[0]], rows_v.at[b],
                                  gsem.at[b]).wait()

        def start_store(j, b):
            pltpu.async_copy(rows_v.at[b],
                             out_hbm.at[pl.ds(row_base + j * CHUNK, CHUNK)],
                             ssem.at[b])

        def wait_store(b):
            pltpu.make_async_copy(
                rows_v.at[b],
                out_hbm.at[pl.ds(row_base, CHUNK)],
                ssem.at[b]).wait()

        for b in range(NBUF):
            start_gather(b, b)

        @pl.loop(0, nchunk - NBUF, step=NBUF)
        def _(g):
            for b in range(NBUF):
                j = g + b
                wait_gather(b)
                start_store(j, b)
                wait_store(b)
                start_gather(j + NBUF, b)

        for b in range(NBUF):
            wait_gather(b)
            start_store(nchunk - NBUF + b, b)
        for b in range(NBUF):
            wait_store(b)

    return emb_kernel


def kernel(x, table):
    num_idx = x.size
    idx = x.reshape(NUM_WORKERS, num_idx // (NUM_WORKERS * CHUNK), CHUNK)
    idx = idx.astype(jnp.int32)
    out = _build(num_idx)(idx, table)
    return out.reshape(x.shape + (EMB_DIM,))
